# baseline (speedup 1.0000x reference)
import jax
import jax.numpy as jnp
from jax import lax
from jax.experimental import pallas as pl
from jax.experimental.pallas import tpu as pltpu

P = 1024
Q = 1024
NH = 512
DECAY = 0.001
TB = 512

_DNT = (((1,), (1,)), ((), ()))


def _dott(x, w):
    return lax.dot_general(x, w, _DNT, preferred_element_type=jnp.float32)


def _softplus(x):
    return jnp.maximum(x, 0.0) + jnp.log(1.0 + jnp.exp(-jnp.abs(x)))


def _celu(x):
    return jnp.where(x > 0.0, x, jnp.exp(jnp.minimum(x, 0.0)) - 1.0)


def _odefunc_kernel(u_ref, aw_ref, ab_ref, fcw_ref, fcb_ref, fhw_ref, fhb_ref,
                    g1w_ref, g1b_ref, g2w_ref, g2b_ref, g3w_ref, g3b_ref,
                    out_ref):
    bf16 = jnp.bfloat16
    u = u_ref[...]
    c = u[:, :P]
    h = u[:, P:]
    cb = c.astype(bf16)
    hb = h.astype(bf16)

    h_ = _softplus(_dott(hb, aw_ref[...]) + ab_ref[...])
    hb_ = h_.astype(bf16)

    fc = _softplus(_dott(cb, fcw_ref[:, :P]) + _dott(hb_, fcw_ref[:, P:])
                   + fcb_ref[...])
    fh = _softplus(_dott(cb, fhw_ref[:, :P]) + _dott(hb_, fhw_ref[:, P:])
                   + fhb_ref[...])
    g = _celu(_dott(cb, g1w_ref[:, :P]) + _dott(hb_, g1w_ref[:, P:])
              + g1b_ref[...])
    g = _celu(_dott(g.astype(bf16), g2w_ref[...]) + g2b_ref[...])
    g = _dott(g.astype(bf16), g3w_ref[...]) + g3b_ref[...]

    out_ref[:, :P] = -fc * c + g - DECAY * c
    out_ref[:, P:] = -fh * h


def kernel(t, u, A_W, A_b, Fc_W, Fc_b, Fh_W, Fh_b,
           Gc_W1, Gc_b1, Gc_W2, Gc_b2, Gc_W3, Gc_b3):
    B = u.shape[0]
    u2 = u.reshape(B, P + Q)

    bf16 = jnp.bfloat16
    aw = A_W.astype(bf16)
    fcw = Fc_W.astype(bf16)
    fhw = Fh_W.astype(bf16)
    g1w = Gc_W1.astype(bf16)
    g2w = Gc_W2.astype(bf16)
    g3w = Gc_W3.astype(bf16)

    grid = (B // TB,)
    inv = lambda i: (0, 0)
    out = pl.pallas_call(
        _odefunc_kernel,
        grid=grid,
        in_specs=[
            pl.BlockSpec((TB, P + Q), lambda i: (i, 0)),
            pl.BlockSpec((Q, Q), inv),
            pl.BlockSpec((1, Q), inv),
            pl.BlockSpec((P, P + Q), inv),
            pl.BlockSpec((1, P), inv),
            pl.BlockSpec((Q, P + Q), inv),
            pl.BlockSpec((1, Q), inv),
            pl.BlockSpec((NH, P + Q), inv),
            pl.BlockSpec((1, NH), inv),
            pl.BlockSpec((NH, NH), inv),
            pl.BlockSpec((1, NH), inv),
            pl.BlockSpec((P, NH), inv),
            pl.BlockSpec((1, P), inv),
        ],
        out_specs=pl.BlockSpec((TB, None, P + Q), lambda i: (i, 0, 0)),
        out_shape=jax.ShapeDtypeStruct((B, 1, P + Q), jnp.float32),
        compiler_params=pltpu.CompilerParams(
            dimension_semantics=("arbitrary",),
        ),
    )(u2, aw, A_b.reshape(1, Q), fcw, Fc_b.reshape(1, P),
      fhw, Fh_b.reshape(1, Q), g1w, Gc_b1.reshape(1, NH),
      g2w, Gc_b2.reshape(1, NH), g3w, Gc_b3.reshape(1, P))

    return out


# speedup vs baseline: 1.2578x; 1.2578x over previous
import jax
import jax.numpy as jnp
from jax import lax
from jax.experimental import pallas as pl
from jax.experimental.pallas import tpu as pltpu

P = 1024
Q = 1024
NH = 512
DECAY = 0.001
TB = 512

_DNT = (((1,), (1,)), ((), ()))


def _dott(x, w):
    return lax.dot_general(x, w, _DNT, preferred_element_type=jnp.float32)


def _softplus(x):
    return jnp.maximum(x, 0.0) + jnp.log(1.0 + jnp.exp(-jnp.abs(x)))


def _celu(x):
    return jnp.where(x > 0.0, x, jnp.exp(jnp.minimum(x, 0.0)) - 1.0)


def _u_copy(u_hbm, ubuf, sems, step, slot):
    return pltpu.make_async_copy(
        u_hbm.at[pl.ds(step * TB, TB), 0, :], ubuf.at[slot], sems.at[slot])


def _odefunc_kernel(u_hbm, aw_ref, ab_ref, fcw_ref, fcb_ref, fhw_ref, fhb_ref,
                    g1w_ref, g1b_ref, g2w_ref, g2b_ref, g3w_ref, g3b_ref,
                    out_ref, ubuf, sems):
    bf16 = jnp.bfloat16
    i = pl.program_id(0)
    n = pl.num_programs(0)
    slot = lax.rem(i, 2)
    nslot = lax.rem(i + 1, 2)

    @pl.when(i == 0)
    def _prologue():
        _u_copy(u_hbm, ubuf, sems, 0, 0).start()

    @pl.when(i + 1 < n)
    def _prefetch():
        _u_copy(u_hbm, ubuf, sems, i + 1, nslot).start()

    _u_copy(u_hbm, ubuf, sems, i, slot).wait()
    u = ubuf[slot]
    c = u[:, :P]
    h = u[:, P:]
    cb = c.astype(bf16)
    hb = h.astype(bf16)

    h_ = _softplus(_dott(hb, aw_ref[...]) + ab_ref[...])
    hb_ = h_.astype(bf16)

    fc = _softplus(_dott(cb, fcw_ref[:, :P]) + _dott(hb_, fcw_ref[:, P:])
                   + fcb_ref[...])
    fh = _softplus(_dott(cb, fhw_ref[:, :P]) + _dott(hb_, fhw_ref[:, P:])
                   + fhb_ref[...])
    g = _celu(_dott(cb, g1w_ref[:, :P]) + _dott(hb_, g1w_ref[:, P:])
              + g1b_ref[...])
    g = _celu(_dott(g.astype(bf16), g2w_ref[...]) + g2b_ref[...])
    g = _dott(g.astype(bf16), g3w_ref[...]) + g3b_ref[...]

    out_ref[:, :P] = -fc * c + g - DECAY * c
    out_ref[:, P:] = -fh * h


def kernel(t, u, A_W, A_b, Fc_W, Fc_b, Fh_W, Fh_b,
           Gc_W1, Gc_b1, Gc_W2, Gc_b2, Gc_W3, Gc_b3):
    B = u.shape[0]

    bf16 = jnp.bfloat16
    aw = A_W.astype(bf16)
    fcw = Fc_W.astype(bf16)
    fhw = Fh_W.astype(bf16)
    g1w = Gc_W1.astype(bf16)
    g2w = Gc_W2.astype(bf16)
    g3w = Gc_W3.astype(bf16)

    grid = (B // TB,)
    inv = lambda i: (0, 0)
    out = pl.pallas_call(
        _odefunc_kernel,
        grid=grid,
        in_specs=[
            pl.BlockSpec(memory_space=pl.ANY),
            pl.BlockSpec((Q, Q), inv),
            pl.BlockSpec((1, Q), inv),
            pl.BlockSpec((P, P + Q), inv),
            pl.BlockSpec((1, P), inv),
            pl.BlockSpec((Q, P + Q), inv),
            pl.BlockSpec((1, Q), inv),
            pl.BlockSpec((NH, P + Q), inv),
            pl.BlockSpec((1, NH), inv),
            pl.BlockSpec((NH, NH), inv),
            pl.BlockSpec((1, NH), inv),
            pl.BlockSpec((P, NH), inv),
            pl.BlockSpec((1, P), inv),
        ],
        out_specs=pl.BlockSpec((TB, None, P + Q), lambda i: (i, 0, 0)),
        out_shape=jax.ShapeDtypeStruct((B, 1, P + Q), jnp.float32),
        scratch_shapes=[
            pltpu.VMEM((2, TB, P + Q), jnp.float32),
            pltpu.SemaphoreType.DMA((2,)),
        ],
        compiler_params=pltpu.CompilerParams(
            dimension_semantics=("arbitrary",),
        ),
    )(u, aw, A_b.reshape(1, Q), fcw, Fc_b.reshape(1, P),
      fhw, Fh_b.reshape(1, Q), g1w, Gc_b1.reshape(1, NH),
      g2w, Gc_b2.reshape(1, NH), g3w, Gc_b3.reshape(1, P))

    return out


# speedup vs baseline: 1.2595x; 1.0013x over previous
import jax
import jax.numpy as jnp
from jax import lax
from jax.experimental import pallas as pl
from jax.experimental.pallas import tpu as pltpu

P = 1024
Q = 1024
NH = 512
DECAY = 0.001
TB = 1024

_DNT = (((1,), (1,)), ((), ()))


def _dott(x, w):
    return lax.dot_general(x, w, _DNT, preferred_element_type=jnp.float32)


def _softplus(x):
    return jnp.maximum(x, 0.0) + jnp.log(1.0 + jnp.exp(-jnp.abs(x)))


def _celu(x):
    return jnp.where(x > 0.0, x, jnp.exp(jnp.minimum(x, 0.0)) - 1.0)


def _u_copy(u_hbm, ubuf, sems, step, slot):
    return pltpu.make_async_copy(
        u_hbm.at[pl.ds(step * TB, TB), 0, :], ubuf.at[slot], sems.at[slot])


def _odefunc_kernel(u_hbm, aw_ref, ab_ref, fcw_ref, fcb_ref, fhw_ref, fhb_ref,
                    g1w_ref, g1b_ref, g2w_ref, g2b_ref, g3w_ref, g3b_ref,
                    out_ref, ubuf, sems):
    bf16 = jnp.bfloat16
    i = pl.program_id(0)
    n = pl.num_programs(0)
    slot = lax.rem(i, 2)
    nslot = lax.rem(i + 1, 2)

    @pl.when(i == 0)
    def _prologue():
        _u_copy(u_hbm, ubuf, sems, 0, 0).start()

    @pl.when(i + 1 < n)
    def _prefetch():
        _u_copy(u_hbm, ubuf, sems, i + 1, nslot).start()

    _u_copy(u_hbm, ubuf, sems, i, slot).wait()
    u = ubuf[slot]
    c = u[:, :P]
    h = u[:, P:]
    cb = c.astype(bf16)
    hb = h.astype(bf16)

    h_ = _softplus(_dott(hb, aw_ref[...]) + ab_ref[...])
    hb_ = h_.astype(bf16)

    fc = _softplus(_dott(cb, fcw_ref[:, :P]) + _dott(hb_, fcw_ref[:, P:])
                   + fcb_ref[...])
    fh = _softplus(_dott(cb, fhw_ref[:, :P]) + _dott(hb_, fhw_ref[:, P:])
                   + fhb_ref[...])
    g = _celu(_dott(cb, g1w_ref[:, :P]) + _dott(hb_, g1w_ref[:, P:])
              + g1b_ref[...])
    g = _celu(_dott(g.astype(bf16), g2w_ref[...]) + g2b_ref[...])
    g = _dott(g.astype(bf16), g3w_ref[...]) + g3b_ref[...]

    out_ref[:, :P] = -fc * c + g - DECAY * c
    out_ref[:, P:] = -fh * h


def kernel(t, u, A_W, A_b, Fc_W, Fc_b, Fh_W, Fh_b,
           Gc_W1, Gc_b1, Gc_W2, Gc_b2, Gc_W3, Gc_b3):
    B = u.shape[0]

    bf16 = jnp.bfloat16
    aw = A_W.astype(bf16)
    fcw = Fc_W.astype(bf16)
    fhw = Fh_W.astype(bf16)
    g1w = Gc_W1.astype(bf16)
    g2w = Gc_W2.astype(bf16)
    g3w = Gc_W3.astype(bf16)

    grid = (B // TB,)
    inv = lambda i: (0, 0)
    out = pl.pallas_call(
        _odefunc_kernel,
        grid=grid,
        in_specs=[
            pl.BlockSpec(memory_space=pl.ANY),
            pl.BlockSpec((Q, Q), inv),
            pl.BlockSpec((1, Q), inv),
            pl.BlockSpec((P, P + Q), inv),
            pl.BlockSpec((1, P), inv),
            pl.BlockSpec((Q, P + Q), inv),
            pl.BlockSpec((1, Q), inv),
            pl.BlockSpec((NH, P + Q), inv),
            pl.BlockSpec((1, NH), inv),
            pl.BlockSpec((NH, NH), inv),
            pl.BlockSpec((1, NH), inv),
            pl.BlockSpec((P, NH), inv),
            pl.BlockSpec((1, P), inv),
        ],
        out_specs=pl.BlockSpec((TB, None, P + Q), lambda i: (i, 0, 0)),
        out_shape=jax.ShapeDtypeStruct((B, 1, P + Q), jnp.float32),
        scratch_shapes=[
            pltpu.VMEM((2, TB, P + Q), jnp.float32),
            pltpu.SemaphoreType.DMA((2,)),
        ],
        compiler_params=pltpu.CompilerParams(
            dimension_semantics=("arbitrary",),
        ),
    )(u, aw, A_b.reshape(1, Q), fcw, Fc_b.reshape(1, P),
      fhw, Fh_b.reshape(1, Q), g1w, Gc_b1.reshape(1, NH),
      g2w, Gc_b2.reshape(1, NH), g3w, Gc_b3.reshape(1, P))

    return out


# speedup vs baseline: 1.4897x; 1.1827x over previous
import jax
import jax.numpy as jnp
from jax import lax
from jax.experimental import pallas as pl
from jax.experimental.pallas import tpu as pltpu

P = 1024
Q = 1024
NH = 512
DECAY = 0.001
TB = 512

_DNT = (((1,), (1,)), ((), ()))


def _dott(x, w):
    return lax.dot_general(x, w, _DNT, preferred_element_type=jnp.float32)


def _softplus(x):
    return jnp.maximum(x, 0.0) + jnp.log(1.0 + jnp.exp(-jnp.abs(x)))


def _celu(x):
    return jnp.where(x > 0.0, x, jnp.exp(jnp.minimum(x, 0.0)) - 1.0)


def _u_copy(u_hbm, ubuf, sems, step, slot):
    return pltpu.make_async_copy(
        u_hbm.at[pl.ds(step * TB, TB), 0, :], ubuf.at[slot], sems.at[slot])


def _odefunc_kernel(u_hbm, aw_hbm, ab_ref, fcw_hbm, fcb_ref, fhw_hbm, fhb_ref,
                    g1w_hbm, g1b_ref, g2w_hbm, g2b_ref, g3w_hbm, g3b_ref,
                    out_ref, ubuf, usem, wstage, wsem,
                    awb, fcwb, fhwb, g1wb, g2wb, g3wb):
    bf16 = jnp.bfloat16
    i = pl.program_id(0)
    n = pl.num_programs(0)
    slot = lax.rem(i, 2)
    nslot = lax.rem(i + 1, 2)

    @pl.when(i == 0)
    def _stage_weights():
        _u_copy(u_hbm, ubuf, usem, 0, 0).start()
        c0 = pltpu.make_async_copy(
            aw_hbm.at[:, pl.ds(0, Q)], wstage.at[0, :Q, :Q], wsem.at[0])
        c0.start()
        c1 = pltpu.make_async_copy(fcw_hbm, wstage.at[1, :P, :], wsem.at[1])
        c1.start()
        c0.wait()
        awb[...] = wstage[0, :Q, :Q].astype(bf16)
        c2 = pltpu.make_async_copy(fhw_hbm, wstage.at[0, :Q, :], wsem.at[0])
        c2.start()
        c1.wait()
        fcwb[...] = wstage[1, :P, :].astype(bf16)
        c3 = pltpu.make_async_copy(g1w_hbm, wstage.at[1, :NH, :], wsem.at[1])
        c3.start()
        c2.wait()
        fhwb[...] = wstage[0, :Q, :].astype(bf16)
        c4 = pltpu.make_async_copy(g2w_hbm, wstage.at[0, :NH, :NH], wsem.at[0])
        c4.start()
        c3.wait()
        g1wb[...] = wstage[1, :NH, :].astype(bf16)
        c5 = pltpu.make_async_copy(g3w_hbm, wstage.at[1, :P, :NH], wsem.at[1])
        c5.start()
        c4.wait()
        g2wb[...] = wstage[0, :NH, :NH].astype(bf16)
        c5.wait()
        g3wb[...] = wstage[1, :P, :NH].astype(bf16)

    @pl.when(i + 1 < n)
    def _prefetch():
        _u_copy(u_hbm, ubuf, usem, i + 1, nslot).start()

    _u_copy(u_hbm, ubuf, usem, i, slot).wait()
    u = ubuf[slot]
    c = u[:, :P]
    h = u[:, P:]
    cb = c.astype(bf16)
    hb = h.astype(bf16)

    h_ = _softplus(_dott(hb, awb[...]) + ab_ref[...])
    hb_ = h_.astype(bf16)

    fc = _softplus(_dott(cb, fcwb[:, :P]) + _dott(hb_, fcwb[:, P:])
                   + fcb_ref[...])
    fh = _softplus(_dott(cb, fhwb[:, :P]) + _dott(hb_, fhwb[:, P:])
                   + fhb_ref[...])
    g = _celu(_dott(cb, g1wb[:, :P]) + _dott(hb_, g1wb[:, P:])
              + g1b_ref[...])
    g = _celu(_dott(g.astype(bf16), g2wb[...]) + g2b_ref[...])
    g = _dott(g.astype(bf16), g3wb[...]) + g3b_ref[...]

    out_ref[:, :P] = -fc * c + g - DECAY * c
    out_ref[:, P:] = -fh * h


def kernel(t, u, A_W, A_b, Fc_W, Fc_b, Fh_W, Fh_b,
           Gc_W1, Gc_b1, Gc_W2, Gc_b2, Gc_W3, Gc_b3):
    B = u.shape[0]
    bf16 = jnp.bfloat16

    grid = (B // TB,)
    inv = lambda i: (0, 0)
    hbm = pl.BlockSpec(memory_space=pl.ANY)
    out = pl.pallas_call(
        _odefunc_kernel,
        grid=grid,
        in_specs=[
            hbm,
            hbm,
            pl.BlockSpec((1, Q), inv),
            hbm,
            pl.BlockSpec((1, P), inv),
            hbm,
            pl.BlockSpec((1, Q), inv),
            hbm,
            pl.BlockSpec((1, NH), inv),
            hbm,
            pl.BlockSpec((1, NH), inv),
            hbm,
            pl.BlockSpec((1, P), inv),
        ],
        out_specs=pl.BlockSpec((TB, None, P + Q), lambda i: (i, 0, 0)),
        out_shape=jax.ShapeDtypeStruct((B, 1, P + Q), jnp.float32),
        scratch_shapes=[
            pltpu.VMEM((2, TB, P + Q), jnp.float32),
            pltpu.SemaphoreType.DMA((2,)),
            pltpu.VMEM((2, P, P + Q), jnp.float32),
            pltpu.SemaphoreType.DMA((2,)),
            pltpu.VMEM((Q, Q), bf16),
            pltpu.VMEM((P, P + Q), bf16),
            pltpu.VMEM((Q, P + Q), bf16),
            pltpu.VMEM((NH, P + Q), bf16),
            pltpu.VMEM((NH, NH), bf16),
            pltpu.VMEM((P, NH), bf16),
        ],
        compiler_params=pltpu.CompilerParams(
            dimension_semantics=("arbitrary",),
        ),
    )(u, A_W, A_b.reshape(1, Q), Fc_W, Fc_b.reshape(1, P),
      Fh_W, Fh_b.reshape(1, Q), Gc_W1, Gc_b1.reshape(1, NH),
      Gc_W2, Gc_b2.reshape(1, NH), Gc_W3, Gc_b3.reshape(1, P))

    return out


# speedup vs baseline: 1.5733x; 1.0561x over previous
import jax
import jax.numpy as jnp
from jax import lax
from jax.experimental import pallas as pl
from jax.experimental.pallas import tpu as pltpu

P = 1024
Q = 1024
NH = 512
DECAY = 0.001
TB = 512

_DNT = (((1,), (1,)), ((), ()))


def _dott(x, w):
    return lax.dot_general(x, w, _DNT, preferred_element_type=jnp.float32)


def _softplus(x):
    return jnp.maximum(x, 0.0) + jnp.log(1.0 + jnp.exp(-jnp.abs(x)))


def _celu(x):
    return jnp.where(x > 0.0, x, jnp.exp(jnp.minimum(x, 0.0)) - 1.0)


def _u_copy(u_hbm, ubuf, sems, step, slot):
    return pltpu.make_async_copy(
        u_hbm.at[pl.ds(step * TB, TB), 0, :], ubuf.at[slot], sems.at[slot])


def _o_copy(out_hbm, obuf, sems, step, slot):
    return pltpu.make_async_copy(
        obuf.at[slot], out_hbm.at[pl.ds(step * TB, TB), 0, :], sems.at[slot])


def _odefunc_kernel(u_hbm, aw_hbm, ab_ref, fcw_hbm, fcb_ref, fhw_hbm, fhb_ref,
                    g1w_hbm, g1b_ref, g2w_hbm, g2b_ref, g3w_hbm, g3b_ref,
                    out_hbm, ubuf, usem, wstage, wsem,
                    awb, fcwb, fhwb, g1wb, g2wb, g3wb, obuf, osem):
    bf16 = jnp.bfloat16
    i = pl.program_id(0)
    n = pl.num_programs(0)
    slot = lax.rem(i, 2)
    nslot = lax.rem(i + 1, 2)

    @pl.when(i == 0)
    def _stage_weights():
        _u_copy(u_hbm, ubuf, usem, 0, 0).start()
        c0 = pltpu.make_async_copy(
            aw_hbm.at[:, pl.ds(0, Q)], wstage.at[0, :Q, :Q], wsem.at[0])
        c0.start()
        c1 = pltpu.make_async_copy(fcw_hbm, wstage.at[1, :P, :], wsem.at[1])
        c1.start()
        c0.wait()
        awb[...] = wstage[0, :Q, :Q].astype(bf16)
        c2 = pltpu.make_async_copy(fhw_hbm, wstage.at[0, :Q, :], wsem.at[0])
        c2.start()
        c1.wait()
        fcwb[...] = wstage[1, :P, :].astype(bf16)
        c3 = pltpu.make_async_copy(g1w_hbm, wstage.at[1, :NH, :], wsem.at[1])
        c3.start()
        c2.wait()
        fhwb[...] = wstage[0, :Q, :].astype(bf16)
        c4 = pltpu.make_async_copy(g2w_hbm, wstage.at[0, :NH, :NH], wsem.at[0])
        c4.start()
        c3.wait()
        g1wb[...] = wstage[1, :NH, :].astype(bf16)
        c5 = pltpu.make_async_copy(g3w_hbm, wstage.at[1, :P, :NH], wsem.at[1])
        c5.start()
        c4.wait()
        g2wb[...] = wstage[0, :NH, :NH].astype(bf16)
        c5.wait()
        g3wb[...] = wstage[1, :P, :NH].astype(bf16)

    @pl.when(i + 1 < n)
    def _prefetch():
        _u_copy(u_hbm, ubuf, usem, i + 1, nslot).start()

    @pl.when(i >= 2)
    def _drain_out():
        _o_copy(out_hbm, obuf, osem, i - 2, slot).wait()

    _u_copy(u_hbm, ubuf, usem, i, slot).wait()
    u = ubuf[slot]
    c = u[:, :P]
    h = u[:, P:]
    cb = c.astype(bf16)
    hb = h.astype(bf16)

    h_ = _softplus(_dott(hb, awb[...]) + ab_ref[...])
    hb_ = h_.astype(bf16)

    fc = _softplus(_dott(cb, fcwb[:, :P]) + _dott(hb_, fcwb[:, P:])
                   + fcb_ref[...])
    fh = _softplus(_dott(cb, fhwb[:, :P]) + _dott(hb_, fhwb[:, P:])
                   + fhb_ref[...])
    g = _celu(_dott(cb, g1wb[:, :P]) + _dott(hb_, g1wb[:, P:])
              + g1b_ref[...])
    g = _celu(_dott(g.astype(bf16), g2wb[...]) + g2b_ref[...])
    g = _dott(g.astype(bf16), g3wb[...]) + g3b_ref[...]

    obuf[slot, :, :P] = -fc * c + g - DECAY * c
    obuf[slot, :, P:] = -fh * h
    _o_copy(out_hbm, obuf, osem, i, slot).start()

    @pl.when(i == n - 1)
    def _epilogue():
        _o_copy(out_hbm, obuf, osem, i - 1, nslot).wait()
        _o_copy(out_hbm, obuf, osem, i, slot).wait()


def kernel(t, u, A_W, A_b, Fc_W, Fc_b, Fh_W, Fh_b,
           Gc_W1, Gc_b1, Gc_W2, Gc_b2, Gc_W3, Gc_b3):
    B = u.shape[0]
    bf16 = jnp.bfloat16

    grid = (B // TB,)
    inv = lambda i: (0, 0)
    hbm = pl.BlockSpec(memory_space=pl.ANY)
    out = pl.pallas_call(
        _odefunc_kernel,
        grid=grid,
        in_specs=[
            hbm,
            hbm,
            pl.BlockSpec((1, Q), inv),
            hbm,
            pl.BlockSpec((1, P), inv),
            hbm,
            pl.BlockSpec((1, Q), inv),
            hbm,
            pl.BlockSpec((1, NH), inv),
            hbm,
            pl.BlockSpec((1, NH), inv),
            hbm,
            pl.BlockSpec((1, P), inv),
        ],
        out_specs=pl.BlockSpec(memory_space=pl.ANY),
        out_shape=jax.ShapeDtypeStruct((B, 1, P + Q), jnp.float32),
        scratch_shapes=[
            pltpu.VMEM((2, TB, P + Q), jnp.float32),
            pltpu.SemaphoreType.DMA((2,)),
            pltpu.VMEM((2, P, P + Q), jnp.float32),
            pltpu.SemaphoreType.DMA((2,)),
            pltpu.VMEM((Q, Q), bf16),
            pltpu.VMEM((P, P + Q), bf16),
            pltpu.VMEM((Q, P + Q), bf16),
            pltpu.VMEM((NH, P + Q), bf16),
            pltpu.VMEM((NH, NH), bf16),
            pltpu.VMEM((P, NH), bf16),
            pltpu.VMEM((2, TB, P + Q), jnp.float32),
            pltpu.SemaphoreType.DMA((2,)),
        ],
        compiler_params=pltpu.CompilerParams(
            dimension_semantics=("arbitrary",),
        ),
    )(u, A_W, A_b.reshape(1, Q), Fc_W, Fc_b.reshape(1, P),
      Fh_W, Fh_b.reshape(1, Q), Gc_W1, Gc_b1.reshape(1, NH),
      Gc_W2, Gc_b2.reshape(1, NH), Gc_W3, Gc_b3.reshape(1, P))

    return out
